# baseline (speedup 1.0000x reference)
import functools

import jax
import jax.numpy as jnp
from jax import lax
from jax.experimental import pallas as pl
from jax.experimental.pallas import tpu as pltpu
from jax.experimental.pallas import tpu_sc as plsc

B = 4096
D = 10000
H = 128
Z = 32
K = 512
BLK = 256
EBLK = 256


def _enc_body(x_ref, w1_ref, b1_ref, w2_ref, b2_ref, e1t_ref, e2t_ref,
              subj_ref, ze_ref, idx_ref):
    x = x_ref[...]
    h = jnp.maximum(
        jnp.dot(x, w1_ref[...], preferred_element_type=jnp.float32)
        + b1_ref[...], 0.0)
    z = jnp.maximum(
        jnp.dot(h, w2_ref[...], preferred_element_type=jnp.float32)
        + b2_ref[...], 0.0)
    ze_ref[...] = z

    def nearest(et):
        d = jnp.zeros((BLK, K), jnp.float32)
        for zi in range(Z):
            diff = z[:, zi:zi + 1] - et[zi:zi + 1, :]
            d = d + diff * diff
        m = jnp.min(d, axis=1, keepdims=True)
        ii = lax.broadcasted_iota(jnp.int32, (BLK, K), 1)
        cand = jnp.where(d == m, ii, K)
        return jnp.min(cand, axis=1)

    k1 = nearest(e1t_ref[...])
    k2 = nearest(e2t_ref[...])
    subj = subj_ref[...][:, 0]
    idx_ref[...] = jnp.where(subj == 0, k1, K + k2)[:, None]


def _table_body(e_ref, wp1_ref, bp1_ref, wp2_ref, bp2_ref, out_ref):
    t = jnp.maximum(
        jnp.dot(e_ref[...], wp1_ref[...], preferred_element_type=jnp.float32)
        + bp1_ref[...], 0.0)
    res = (jnp.dot(t, wp2_ref[...], preferred_element_type=jnp.float32)
           + bp2_ref[...])
    out_ref[...] = jnp.concatenate(
        [res, jnp.zeros((EBLK, 128 - Z), jnp.float32)], axis=1)


def _make_sc_gather(n_rows, n_cols, n_batch, num_cores, num_subcores):
    nw = num_cores * num_subcores
    b_per_w = n_batch // nw
    mesh = plsc.VectorSubcoreMesh(core_axis_name="c", subcore_axis_name="s")

    @functools.partial(
        pl.kernel, mesh=mesh,
        out_type=jax.ShapeDtypeStruct((n_batch, n_cols), jnp.float32),
        scratch_types=[
            pltpu.VMEM((b_per_w,), jnp.int32),
            pltpu.VMEM((b_per_w, n_cols), jnp.float32),
            pltpu.SemaphoreType.DMA,
        ],
    )
    def gather(table_hbm, idx_hbm, out_hbm, idx_v, rows_v, sem):
        wid = lax.axis_index("s") * num_cores + lax.axis_index("c")
        base = wid * b_per_w
        pltpu.sync_copy(idx_hbm.at[pl.ds(base, b_per_w)], idx_v)
        pltpu.async_copy(table_hbm.at[idx_v], rows_v, sem).wait()
        pltpu.sync_copy(rows_v, out_hbm.at[pl.ds(base, b_per_w)])

    return gather


def _enc_half(off, data, W1, b1, W2, b2, e1t, e2t, subj2d):
    nb = B // (2 * BLK)
    return pl.pallas_call(
        _enc_body,
        grid=(nb,),
        in_specs=[
            pl.BlockSpec((BLK, D), lambda i: (i + off, 0)),
            pl.BlockSpec((D, H), lambda i: (0, 0)),
            pl.BlockSpec((1, H), lambda i: (0, 0)),
            pl.BlockSpec((H, Z), lambda i: (0, 0)),
            pl.BlockSpec((1, Z), lambda i: (0, 0)),
            pl.BlockSpec((Z, K), lambda i: (0, 0)),
            pl.BlockSpec((Z, K), lambda i: (0, 0)),
            pl.BlockSpec((BLK, 1), lambda i: (i + off, 0)),
        ],
        out_specs=[
            pl.BlockSpec((BLK, Z), lambda i: (i, 0)),
            pl.BlockSpec((BLK, 1), lambda i: (i, 0)),
        ],
        out_shape=[
            jax.ShapeDtypeStruct((B // 2, Z), jnp.float32),
            jax.ShapeDtypeStruct((B // 2, 1), jnp.int32),
        ],
    )(data, W1, b1, W2, b2, e1t, e2t, subj2d)


def kernel(data, subject, W1, b1, W2, b2, embeddings_1, embeddings_2,
           Wp1, bp1, Wp2, bp2):
    e1t = embeddings_1.T
    e2t = embeddings_2.T
    b1r = b1.reshape(1, H)
    b2r = b2.reshape(1, Z)
    subj2d = subject.reshape(B, 1).astype(jnp.int32)
    nb = B // (2 * BLK)

    E = jnp.concatenate([embeddings_1, embeddings_2], axis=0)
    table = pl.pallas_call(
        _table_body,
        grid=(2 * K // EBLK,),
        in_specs=[
            pl.BlockSpec((EBLK, Z), lambda i: (i, 0)),
            pl.BlockSpec((Z, D), lambda i: (0, 0)),
            pl.BlockSpec((1, D), lambda i: (0, 0)),
            pl.BlockSpec((D, Z), lambda i: (0, 0)),
            pl.BlockSpec((1, Z), lambda i: (0, 0)),
        ],
        out_specs=pl.BlockSpec((EBLK, 128), lambda i: (i, 0)),
        out_shape=jax.ShapeDtypeStruct((2 * K, 128), jnp.float32),
    )(E, Wp1, bp1.reshape(1, D), Wp2, bp2.reshape(1, Z))

    info = plsc.get_sparse_core_info()
    gat = _make_sc_gather(2 * K, 128, B // 2,
                          info.num_cores, info.num_subcores)
    z_e0, idx0 = _enc_half(0, data, W1, b1r, W2, b2r, e1t, e2t, subj2d)
    z_c0 = gat(table, idx0.reshape(B // 2))
    z_e1, idx1 = _enc_half(nb, data, W1, b1r, W2, b2r, e1t, e2t, subj2d)
    z_c1 = gat(table, idx1.reshape(B // 2))
    z_e = jnp.concatenate([z_e0, z_e1], axis=0)
    z_c = jnp.concatenate([z_c0[:, :Z], z_c1[:, :Z]], axis=0)
    return (z_e, z_c)


# speedup vs baseline: 1.0049x; 1.0049x over previous
import functools

import jax
import jax.numpy as jnp
from jax import lax
from jax.experimental import pallas as pl
from jax.experimental.pallas import tpu as pltpu
from jax.experimental.pallas import tpu_sc as plsc

B = 4096
D = 10000
H = 128
Z = 32
K = 512
BLK = 256
EBLK = 256


def _enc_body(x_ref, w1_ref, b1_ref, w2_ref, b2_ref, ecat_t_ref,
              subj_ref, ze_ref, idx_ref):
    x = x_ref[...]
    h = jnp.maximum(
        jnp.dot(x, w1_ref[...], preferred_element_type=jnp.float32)
        + b1_ref[...], 0.0)
    z = jnp.maximum(
        jnp.dot(h, w2_ref[...], preferred_element_type=jnp.float32)
        + b2_ref[...], 0.0)
    ze_ref[...] = z

    et = ecat_t_ref[...]
    d = jnp.zeros((BLK, 2 * K), jnp.float32)
    for zi in range(Z):
        diff = z[:, zi:zi + 1] - et[zi:zi + 1, :]
        d = d + diff * diff
    ii = lax.broadcasted_iota(jnp.int32, (BLK, K), 1)

    def amin(dc):
        m = jnp.min(dc, axis=1, keepdims=True)
        return jnp.min(jnp.where(dc == m, ii, K), axis=1)

    k1 = amin(d[:, :K])
    k2 = amin(d[:, K:])
    subj = subj_ref[...][:, 0]
    idx_ref[...] = jnp.where(subj == 0, k1, K + k2)[:, None]


def _table_body(e_ref, wp1_ref, bp1_ref, wp2_ref, bp2_ref, out_ref):
    t = jnp.maximum(
        jnp.dot(e_ref[...], wp1_ref[...], preferred_element_type=jnp.float32)
        + bp1_ref[...], 0.0)
    res = (jnp.dot(t, wp2_ref[...], preferred_element_type=jnp.float32)
           + bp2_ref[...])
    out_ref[...] = jnp.concatenate(
        [res, jnp.zeros((EBLK, 128 - Z), jnp.float32)], axis=1)


def _make_sc_gather(n_rows, n_cols, n_batch, num_cores, num_subcores):
    nw = num_cores * num_subcores
    b_per_w = n_batch // nw
    mesh = plsc.VectorSubcoreMesh(core_axis_name="c", subcore_axis_name="s")

    @functools.partial(
        pl.kernel, mesh=mesh,
        out_type=jax.ShapeDtypeStruct((n_batch, n_cols), jnp.float32),
        scratch_types=[
            pltpu.VMEM((b_per_w,), jnp.int32),
            pltpu.VMEM((b_per_w, n_cols), jnp.float32),
            pltpu.SemaphoreType.DMA,
        ],
    )
    def gather(table_hbm, idx_hbm, out_hbm, idx_v, rows_v, sem):
        wid = lax.axis_index("s") * num_cores + lax.axis_index("c")
        base = wid * b_per_w
        pltpu.sync_copy(idx_hbm.at[pl.ds(base, b_per_w)], idx_v)
        pltpu.async_copy(table_hbm.at[idx_v], rows_v, sem).wait()
        pltpu.sync_copy(rows_v, out_hbm.at[pl.ds(base, b_per_w)])

    return gather


def kernel(data, subject, W1, b1, W2, b2, embeddings_1, embeddings_2,
           Wp1, bp1, Wp2, bp2):
    z_e, idx = pl.pallas_call(
        _enc_body,
        grid=(B // BLK,),
        in_specs=[
            pl.BlockSpec((BLK, D), lambda i: (i, 0)),
            pl.BlockSpec((D, H), lambda i: (0, 0)),
            pl.BlockSpec((1, H), lambda i: (0, 0)),
            pl.BlockSpec((H, Z), lambda i: (0, 0)),
            pl.BlockSpec((1, Z), lambda i: (0, 0)),
            pl.BlockSpec((Z, 2 * K), lambda i: (0, 0)),
            pl.BlockSpec((BLK, 1), lambda i: (i, 0)),
        ],
        out_specs=[
            pl.BlockSpec((BLK, Z), lambda i: (i, 0)),
            pl.BlockSpec((BLK, 1), lambda i: (i, 0)),
        ],
        out_shape=[
            jax.ShapeDtypeStruct((B, Z), jnp.float32),
            jax.ShapeDtypeStruct((B, 1), jnp.int32),
        ],
    )(data, W1, b1.reshape(1, H), W2, b2.reshape(1, Z),
      jnp.concatenate([embeddings_1, embeddings_2], axis=0).T,
      subject.reshape(B, 1).astype(jnp.int32))

    E = jnp.concatenate([embeddings_1, embeddings_2], axis=0)
    table = pl.pallas_call(
        _table_body,
        grid=(2 * K // EBLK,),
        in_specs=[
            pl.BlockSpec((EBLK, Z), lambda i: (i, 0)),
            pl.BlockSpec((Z, D), lambda i: (0, 0)),
            pl.BlockSpec((1, D), lambda i: (0, 0)),
            pl.BlockSpec((D, Z), lambda i: (0, 0)),
            pl.BlockSpec((1, Z), lambda i: (0, 0)),
        ],
        out_specs=pl.BlockSpec((EBLK, 128), lambda i: (i, 0)),
        out_shape=jax.ShapeDtypeStruct((2 * K, 128), jnp.float32),
    )(E, Wp1, bp1.reshape(1, D), Wp2, bp2.reshape(1, Z))

    info = plsc.get_sparse_core_info()
    z_c_pad = _make_sc_gather(2 * K, 128, B, info.num_cores, info.num_subcores)(
        table, idx.reshape(B))
    return (z_e, z_c_pad[:, :Z])


# speedup vs baseline: 1.0204x; 1.0154x over previous
import functools

import jax
import jax.numpy as jnp
from jax import lax
from jax.experimental import pallas as pl
from jax.experimental.pallas import tpu as pltpu
from jax.experimental.pallas import tpu_sc as plsc

B = 4096
D = 10000
H = 128
Z = 32
K = 512
BLK = 256
EBLK = 256


def _enc_body(x_ref, w1_ref, b1_ref, w2_ref, b2_ref, e1t_ref, e2t_ref,
              subj_ref, ze_ref, idx_ref):
    x = x_ref[...]
    h = jnp.maximum(
        jnp.dot(x, w1_ref[...], preferred_element_type=jnp.float32)
        + b1_ref[...], 0.0)
    z = jnp.maximum(
        jnp.dot(h, w2_ref[...], preferred_element_type=jnp.float32)
        + b2_ref[...], 0.0)
    ze_ref[...] = z

    def nearest(et):
        d = jnp.zeros((BLK, K), jnp.float32)
        for zi in range(Z):
            diff = z[:, zi:zi + 1] - et[zi:zi + 1, :]
            d = d + diff * diff
        m = jnp.min(d, axis=1, keepdims=True)
        ii = lax.broadcasted_iota(jnp.int32, (BLK, K), 1)
        cand = jnp.where(d == m, ii, K)
        return jnp.min(cand, axis=1)

    k1 = nearest(e1t_ref[...])
    k2 = nearest(e2t_ref[...])
    subj = subj_ref[...][:, 0]
    idx_ref[...] = jnp.where(subj == 0, k1, K + k2)[:, None]


def _table_body(e_ref, wp1_ref, bp1_ref, wp2_ref, bp2_ref, out_ref):
    t = jnp.maximum(
        jnp.dot(e_ref[...], wp1_ref[...], preferred_element_type=jnp.float32)
        + bp1_ref[...], 0.0)
    res = (jnp.dot(t, wp2_ref[...], preferred_element_type=jnp.float32)
           + bp2_ref[...])
    out_ref[...] = jnp.concatenate(
        [res, jnp.zeros((EBLK, 128 - Z), jnp.float32)], axis=1)


def _make_sc_gather(n_rows, n_cols, n_batch, num_cores, num_subcores):
    nw = num_cores * num_subcores
    b_per_w = n_batch // nw
    mesh = plsc.VectorSubcoreMesh(core_axis_name="c", subcore_axis_name="s")

    @functools.partial(
        pl.kernel, mesh=mesh,
        out_type=jax.ShapeDtypeStruct((n_batch, n_cols), jnp.float32),
        scratch_types=[
            pltpu.VMEM((b_per_w,), jnp.int32),
            pltpu.VMEM((b_per_w, n_cols), jnp.float32),
            pltpu.SemaphoreType.DMA,
        ],
    )
    def gather(table_hbm, idx_hbm, out_hbm, idx_v, rows_v, sem):
        wid = lax.axis_index("s") * num_cores + lax.axis_index("c")
        base = wid * b_per_w
        pltpu.sync_copy(idx_hbm.at[pl.ds(base, b_per_w)], idx_v)
        pltpu.async_copy(table_hbm.at[idx_v], rows_v, sem).wait()
        pltpu.sync_copy(rows_v, out_hbm.at[pl.ds(base, b_per_w)])

    return gather


def kernel(data, subject, W1, b1, W2, b2, embeddings_1, embeddings_2,
           Wp1, bp1, Wp2, bp2):
    z_e, idx = pl.pallas_call(
        _enc_body,
        grid=(B // BLK,),
        in_specs=[
            pl.BlockSpec((BLK, D), lambda i: (i, 0)),
            pl.BlockSpec((D, H), lambda i: (0, 0)),
            pl.BlockSpec((1, H), lambda i: (0, 0)),
            pl.BlockSpec((H, Z), lambda i: (0, 0)),
            pl.BlockSpec((1, Z), lambda i: (0, 0)),
            pl.BlockSpec((Z, K), lambda i: (0, 0)),
            pl.BlockSpec((Z, K), lambda i: (0, 0)),
            pl.BlockSpec((BLK, 1), lambda i: (i, 0)),
        ],
        out_specs=[
            pl.BlockSpec((BLK, Z), lambda i: (i, 0)),
            pl.BlockSpec((BLK, 1), lambda i: (i, 0)),
        ],
        out_shape=[
            jax.ShapeDtypeStruct((B, Z), jnp.float32),
            jax.ShapeDtypeStruct((B, 1), jnp.int32),
        ],
    )(data, W1, b1.reshape(1, H), W2, b2.reshape(1, Z),
      embeddings_1.T, embeddings_2.T,
      subject.reshape(B, 1).astype(jnp.int32))

    E = jnp.concatenate([embeddings_1, embeddings_2], axis=0)
    table = pl.pallas_call(
        _table_body,
        grid=(2 * K // EBLK,),
        in_specs=[
            pl.BlockSpec((EBLK, Z), lambda i: (i, 0)),
            pl.BlockSpec((Z, D), lambda i: (0, 0)),
            pl.BlockSpec((1, D), lambda i: (0, 0)),
            pl.BlockSpec((D, Z), lambda i: (0, 0)),
            pl.BlockSpec((1, Z), lambda i: (0, 0)),
        ],
        out_specs=pl.BlockSpec((EBLK, 128), lambda i: (i, 0)),
        out_shape=jax.ShapeDtypeStruct((2 * K, 128), jnp.float32),
    )(E, Wp1, bp1.reshape(1, D), Wp2, bp2.reshape(1, Z))

    info = plsc.get_sparse_core_info()
    z_c_pad = _make_sc_gather(2 * K, 128, B, info.num_cores, info.num_subcores)(
        table, idx.reshape(B))
    return (z_e, z_c_pad[:, :Z])
